# Initial kernel scaffold; baseline (speedup 1.0000x reference)
import jax
import jax.numpy as jnp
from jax.experimental import pallas as pl


def kernel(features, tables):
    raise NotImplementedError("write your pallas kernel here")


# speedup vs baseline: 1.2058x; 1.2058x over previous
import functools

import jax
import jax.numpy as jnp
from jax.experimental import pallas as pl
from jax.experimental.pallas import tpu as pltpu
from jax.experimental.pallas import tpu_sc as plsc

_F = 26
_V = 100000
_D = 32
_B = 16384
_N = _B * _F
_W = 512


def _sc_gather(flat_table, flat_idx):
  mesh = plsc.VectorSubcoreMesh(
      core_axis_name="core", subcore_axis_name="subcore")

  @functools.partial(
      pl.kernel,
      out_type=jax.ShapeDtypeStruct((_N, _D), jnp.float32),
      mesh=mesh,
      compiler_params=pltpu.CompilerParams(use_tc_tiling_on_sc=False),
  )
  def k(table_hbm, idx_hbm, out_hbm):
    def body(i_vmem, o_vmem):
      pltpu.sync_copy(table_hbm.at[i_vmem.at[0]], o_vmem)

    pltpu.emit_pipeline(
        body,
        grid=(_N // _W,),
        in_specs=[pl.BlockSpec((1, _W), index_map=lambda i: (0, i))],
        out_specs=[pl.BlockSpec((_W, _D), index_map=lambda i: (i, 0))],
        core_axis_name=("core", "subcore"),
        dimension_semantics=(pltpu.PARALLEL,),
    )(idx_hbm, out_hbm)

  return k(flat_table, flat_idx)


@jax.jit
def kernel(features, tables):
  flat_idx = features.astype(jnp.int32) + (
      jnp.arange(_F, dtype=jnp.int32) * _V)[None, :]
  flat_idx = flat_idx.reshape(1, _N)
  flat_table = tables.reshape(_F * _V, _D)
  out = _sc_gather(flat_table, flat_idx)
  return out.reshape(_B, _F * _D)


# speedup vs baseline: 1.2111x; 1.0045x over previous
import functools

import jax
import jax.numpy as jnp
from jax.experimental import pallas as pl
from jax.experimental.pallas import tpu as pltpu
from jax.experimental.pallas import tpu_sc as plsc

_F = 26
_V = 100000
_D = 32
_B = 16384
_N = _B * _F
_W = 1024


def _sc_gather(flat_table, flat_idx):
  mesh = plsc.VectorSubcoreMesh(
      core_axis_name="core", subcore_axis_name="subcore")

  @functools.partial(
      pl.kernel,
      out_type=jax.ShapeDtypeStruct((_N, _D), jnp.float32),
      mesh=mesh,
      compiler_params=pltpu.CompilerParams(use_tc_tiling_on_sc=False),
  )
  def k(table_hbm, idx_hbm, out_hbm):
    def body(i_vmem, o_vmem):
      pltpu.sync_copy(table_hbm.at[i_vmem.at[0]], o_vmem)

    pltpu.emit_pipeline(
        body,
        grid=(_N // _W,),
        in_specs=[pl.BlockSpec((1, _W), index_map=lambda i: (0, i))],
        out_specs=[pl.BlockSpec((_W, _D), index_map=lambda i: (i, 0))],
        core_axis_name=("core", "subcore"),
        dimension_semantics=(pltpu.PARALLEL,),
    )(idx_hbm, out_hbm)

  return k(flat_table, flat_idx)


@jax.jit
def kernel(features, tables):
  flat_idx = features.astype(jnp.int32) + (
      jnp.arange(_F, dtype=jnp.int32) * _V)[None, :]
  flat_idx = flat_idx.reshape(1, _N)
  flat_table = tables.reshape(_F * _V, _D)
  out = _sc_gather(flat_table, flat_idx)
  return out.reshape(_B, _F * _D)
